# baseline (device time: 17653 ns/iter reference)
import jax
import jax.numpy as jnp
from jax import lax
from jax.experimental import pallas as pl
from jax.experimental.pallas import tpu as pltpu

K = 16
FOLD_LEVELS = 4
KS = [16, 8, 4, 2, 2]
N_GROUP = 8
_NEG = -3.0e38


def _fold(arrs):
    new = []
    for m in range(len(arrs) + 1):
        parts = []
        if m < len(arrs):
            a = arrs[m]
            w = a.shape[1] // 2
            parts.append(jnp.maximum(a[:, :w], a[:, w:]))
        if m >= 1:
            a = arrs[m - 1]
            w = a.shape[1] // 2
            parts.append(jnp.minimum(a[:, :w], a[:, w:]))
        new.append(parts[0] if len(parts) == 1 else jnp.concatenate(parts, axis=1))
    return new


def _topk_cols(vals, k):
    outs = []
    for _ in range(k):
        m = jnp.max(vals, axis=1, keepdims=True)
        outs.append(m)
        vals = jnp.where(vals == m, _NEG, vals)
    return outs


def kernel(x):
    m, n = x.shape
    rows = m // N_GROUP

    def body(x_ref, out_ref, x_slice, send_buf, recv_buf, copy_sem,
             sem_x_send, sem_x_recv, gather_send_sems, gather_recv_sems):
        my_x = lax.axis_index("x")
        my_y = lax.axis_index("y")
        my_z = lax.axis_index("z")
        partner = (1 - my_x, my_y, my_z)
        gid = my_y * 4 + my_z
        row0 = gid * rows

        slice_copy = pltpu.make_async_copy(
            x_ref.at[pl.ds(row0, rows), :], x_slice, copy_sem
        )
        slice_copy.start()

        barrier_sem = pltpu.get_barrier_semaphore()
        pl.semaphore_signal(
            barrier_sem, inc=1, device_id=partner,
            device_id_type=pl.DeviceIdType.MESH,
        )
        for py in range(2):
            for pz in range(4):
                g = py * 4 + pz
                is_self = jnp.logical_and(py == my_y, pz == my_z)

                @pl.when(jnp.logical_not(is_self))
                def _():
                    pl.semaphore_signal(
                        barrier_sem, inc=1, device_id=(my_x, py, pz),
                        device_id_type=pl.DeviceIdType.MESH,
                    )
        pl.semaphore_wait(barrier_sem, N_GROUP - 1 + 1)

        slice_copy.wait()
        arrs = [x_slice[:, :]]
        for _ in range(FOLD_LEVELS):
            arrs = _fold(arrs)
        cand = []
        for a, k in zip(arrs, KS):
            cand.extend(_topk_cols(a, k))
        local = jnp.concatenate(cand, axis=1)
        send_buf[:, :] = local

        rdma = pltpu.make_async_remote_copy(
            src_ref=send_buf,
            dst_ref=recv_buf,
            send_sem=sem_x_send,
            recv_sem=sem_x_recv,
            device_id=partner,
            device_id_type=pl.DeviceIdType.MESH,
        )
        rdma.start()
        rdma.wait()

        both = jnp.concatenate([local, recv_buf[:, :]], axis=1)
        merged = jnp.concatenate(_topk_cols(both, K), axis=1)
        out_ref[pl.ds(row0, rows), :] = merged

        pushes = []
        for py in range(2):
            for pz in range(4):
                g = py * 4 + pz
                is_self = jnp.logical_and(py == my_y, pz == my_z)

                @pl.when(jnp.logical_not(is_self))
                def _(py=py, pz=pz, g=g):
                    push = pltpu.make_async_remote_copy(
                        src_ref=out_ref.at[pl.ds(row0, rows), :],
                        dst_ref=out_ref.at[pl.ds(row0, rows), :],
                        send_sem=gather_send_sems.at[g],
                        recv_sem=gather_recv_sems.at[gid],
                        device_id=(my_x, py, pz),
                        device_id_type=pl.DeviceIdType.MESH,
                    )
                    push.start()

        for py in range(2):
            for pz in range(4):
                g = py * 4 + pz
                is_self = jnp.logical_and(py == my_y, pz == my_z)

                @pl.when(jnp.logical_not(is_self))
                def _(py=py, pz=pz, g=g):
                    peer_row0 = g * rows
                    recv = pltpu.make_async_remote_copy(
                        src_ref=out_ref.at[pl.ds(peer_row0, rows), :],
                        dst_ref=out_ref.at[pl.ds(peer_row0, rows), :],
                        send_sem=gather_send_sems.at[g],
                        recv_sem=gather_recv_sems.at[g],
                        device_id=(my_x, py, pz),
                        device_id_type=pl.DeviceIdType.MESH,
                    )
                    recv.wait_recv()
                    send = pltpu.make_async_remote_copy(
                        src_ref=out_ref.at[pl.ds(row0, rows), :],
                        dst_ref=out_ref.at[pl.ds(row0, rows), :],
                        send_sem=gather_send_sems.at[g],
                        recv_sem=gather_recv_sems.at[gid],
                        device_id=(my_x, py, pz),
                        device_id_type=pl.DeviceIdType.MESH,
                    )
                    send.wait_send()

    return pl.pallas_call(
        body,
        out_shape=jax.ShapeDtypeStruct((m, K), jnp.float32),
        in_specs=[pl.BlockSpec(memory_space=pl.ANY)],
        out_specs=pl.BlockSpec(memory_space=pltpu.VMEM),
        scratch_shapes=[
            pltpu.VMEM((rows, n), jnp.float32),
            pltpu.VMEM((rows, 2 * K), jnp.float32),
            pltpu.VMEM((rows, 2 * K), jnp.float32),
            pltpu.SemaphoreType.DMA,
            pltpu.SemaphoreType.DMA,
            pltpu.SemaphoreType.DMA,
            pltpu.SemaphoreType.DMA((N_GROUP,)),
            pltpu.SemaphoreType.DMA((N_GROUP,)),
        ],
        compiler_params=pltpu.CompilerParams(collective_id=0),
    )(x)


# device time: 16485 ns/iter; 1.0709x vs baseline; 1.0709x over previous
import jax
import jax.numpy as jnp
from jax import lax
from jax.experimental import pallas as pl
from jax.experimental.pallas import tpu as pltpu

K = 16
FOLD_LEVELS = 4
KS = [16, 8, 4, 2, 2]
N_GROUP = 8
_NEG = -3.0e38


def _fold(arrs):
    new = []
    for m in range(len(arrs) + 1):
        parts = []
        if m < len(arrs):
            a = arrs[m]
            w = a.shape[1] // 2
            parts.append(jnp.maximum(a[:, :w], a[:, w:]))
        if m >= 1:
            a = arrs[m - 1]
            w = a.shape[1] // 2
            parts.append(jnp.minimum(a[:, :w], a[:, w:]))
        new.append(parts[0] if len(parts) == 1 else jnp.concatenate(parts, axis=1))
    return new


def _topk_cols(vals, k):
    outs = []
    for _ in range(k):
        m = jnp.max(vals, axis=1, keepdims=True)
        outs.append(m)
        vals = jnp.where(vals == m, _NEG, vals)
    return outs


def kernel(x):
    m, n = x.shape
    rows = m // N_GROUP

    def body(x_ref, out_ref, x_slice, send_buf, recv_buf, copy_sem,
             sem_x_send, sem_x_recv, gather_send_sems, gather_recv_sems,
             group_sem):
        my_x = lax.axis_index("x")
        my_y = lax.axis_index("y")
        my_z = lax.axis_index("z")
        partner = (1 - my_x, my_y, my_z)
        gid = my_y * 4 + my_z
        row0 = gid * rows

        slice_copy = pltpu.make_async_copy(
            x_ref.at[pl.ds(row0, rows), :], x_slice, copy_sem
        )
        slice_copy.start()

        barrier_sem = pltpu.get_barrier_semaphore()
        pl.semaphore_signal(
            barrier_sem, inc=1, device_id=partner,
            device_id_type=pl.DeviceIdType.MESH,
        )
        for py in range(2):
            for pz in range(4):
                is_self = jnp.logical_and(py == my_y, pz == my_z)

                @pl.when(jnp.logical_not(is_self))
                def _():
                    pl.semaphore_signal(
                        group_sem, inc=1, device_id=(my_x, py, pz),
                        device_id_type=pl.DeviceIdType.MESH,
                    )

        slice_copy.wait()
        arrs = [x_slice[:, :]]
        for _ in range(FOLD_LEVELS):
            arrs = _fold(arrs)
        cand = []
        for a, k in zip(arrs, KS):
            cand.extend(_topk_cols(a, k))
        local = jnp.concatenate(cand, axis=1)
        send_buf[:, :] = local

        pl.semaphore_wait(barrier_sem, 1)
        rdma = pltpu.make_async_remote_copy(
            src_ref=send_buf,
            dst_ref=recv_buf,
            send_sem=sem_x_send,
            recv_sem=sem_x_recv,
            device_id=partner,
            device_id_type=pl.DeviceIdType.MESH,
        )
        rdma.start()
        pl.semaphore_wait(group_sem, N_GROUP - 1)
        rdma.wait_recv()

        both = jnp.concatenate([local, recv_buf[:, :]], axis=1)
        merged = jnp.concatenate(_topk_cols(both, K), axis=1)
        out_ref[pl.ds(row0, rows), :] = merged

        for py in range(2):
            for pz in range(4):
                g = py * 4 + pz
                is_self = jnp.logical_and(py == my_y, pz == my_z)

                @pl.when(jnp.logical_not(is_self))
                def _(py=py, pz=pz, g=g):
                    push = pltpu.make_async_remote_copy(
                        src_ref=out_ref.at[pl.ds(row0, rows), :],
                        dst_ref=out_ref.at[pl.ds(row0, rows), :],
                        send_sem=gather_send_sems.at[g],
                        recv_sem=gather_recv_sems.at[gid],
                        device_id=(my_x, py, pz),
                        device_id_type=pl.DeviceIdType.MESH,
                    )
                    push.start()

        for py in range(2):
            for pz in range(4):
                g = py * 4 + pz
                is_self = jnp.logical_and(py == my_y, pz == my_z)

                @pl.when(jnp.logical_not(is_self))
                def _(py=py, pz=pz, g=g):
                    peer_row0 = g * rows
                    recv = pltpu.make_async_remote_copy(
                        src_ref=out_ref.at[pl.ds(peer_row0, rows), :],
                        dst_ref=out_ref.at[pl.ds(peer_row0, rows), :],
                        send_sem=gather_send_sems.at[g],
                        recv_sem=gather_recv_sems.at[g],
                        device_id=(my_x, py, pz),
                        device_id_type=pl.DeviceIdType.MESH,
                    )
                    recv.wait_recv()
                    send = pltpu.make_async_remote_copy(
                        src_ref=out_ref.at[pl.ds(row0, rows), :],
                        dst_ref=out_ref.at[pl.ds(row0, rows), :],
                        send_sem=gather_send_sems.at[g],
                        recv_sem=gather_recv_sems.at[gid],
                        device_id=(my_x, py, pz),
                        device_id_type=pl.DeviceIdType.MESH,
                    )
                    send.wait_send()

        rdma.wait_send()

    return pl.pallas_call(
        body,
        out_shape=jax.ShapeDtypeStruct((m, K), jnp.float32),
        in_specs=[pl.BlockSpec(memory_space=pl.ANY)],
        out_specs=pl.BlockSpec(memory_space=pltpu.VMEM),
        scratch_shapes=[
            pltpu.VMEM((rows, n), jnp.float32),
            pltpu.VMEM((rows, 2 * K), jnp.float32),
            pltpu.VMEM((rows, 2 * K), jnp.float32),
            pltpu.SemaphoreType.DMA,
            pltpu.SemaphoreType.DMA,
            pltpu.SemaphoreType.DMA,
            pltpu.SemaphoreType.DMA((N_GROUP,)),
            pltpu.SemaphoreType.DMA((N_GROUP,)),
            pltpu.SemaphoreType.REGULAR,
        ],
        compiler_params=pltpu.CompilerParams(collective_id=0),
    )(x)
